# Initial kernel scaffold; baseline (speedup 1.0000x reference)
import jax
import jax.numpy as jnp
from jax.experimental import pallas as pl


def kernel(x, edge_index, W, att_src, att_dst, bias):
    raise NotImplementedError("write your pallas kernel here")


# speedup vs baseline: 23.7503x; 23.7503x over previous
import functools

import jax
import jax.numpy as jnp
from jax import lax
from jax.experimental import pallas as pl
from jax.experimental.pallas import tpu as pltpu
from jax.experimental.pallas import tpu_sc as plsc

N = 10000
E = 320000
D = 128

NC = 2
NS = 16
NW = NC * NS
EPW = E // NW
K = 80
NCHUNK = EPW // K


def _tc1_body(x_ref, w_ref, as_ref, ad_ref, h_ref, a8_ref, m2_ref):
    i = pl.program_id(0)
    h = jnp.dot(x_ref[...], w_ref[...], preferred_element_type=jnp.float32)
    h_ref[...] = h
    a_s = jnp.sum(h * as_ref[...], axis=1)
    a_d = jnp.sum(h * ad_ref[...], axis=1)
    z = jnp.zeros_like(a_s)
    a8_ref[...] = jnp.stack([a_s, a_d, z, z, z, z, z, z], axis=1)
    mblk = jnp.stack([jnp.full((16,), jnp.max(a_s)),
                      jnp.full((16,), jnp.max(a_d))], axis=0)

    @pl.when(i == 0)
    def _():
        m2_ref[...] = mblk

    @pl.when(i > 0)
    def _():
        m2_ref[...] = jnp.maximum(m2_ref[...], mblk)


def _tc1(x, W, att_src, att_dst):
    blk = 1000
    return pl.pallas_call(
        _tc1_body,
        grid=(N // blk,),
        in_specs=[
            pl.BlockSpec((blk, D), lambda i: (i, 0)),
            pl.BlockSpec((D, D), lambda i: (0, 0)),
            pl.BlockSpec((1, D), lambda i: (0, 0)),
            pl.BlockSpec((1, D), lambda i: (0, 0)),
        ],
        out_specs=[
            pl.BlockSpec((blk, D), lambda i: (i, 0)),
            pl.BlockSpec((blk, 8), lambda i: (i, 0)),
            pl.BlockSpec((2, 16), lambda i: (0, 0)),
        ],
        out_shape=[
            jax.ShapeDtypeStruct((N, D), jnp.float32),
            jax.ShapeDtypeStruct((N, 8), jnp.float32),
            jax.ShapeDtypeStruct((2, 16), jnp.float32),
        ],
    )(x, W, att_src.reshape(1, D), att_dst.reshape(1, D))


def _sc1(src, dst, a_src, a_dst, m2):
    mesh = plsc.VectorSubcoreMesh(core_axis_name="c", subcore_axis_name="s")

    @functools.partial(
        pl.kernel,
        mesh=mesh,
        compiler_params=pltpu.CompilerParams(needs_layout_passes=False),
        out_type=[
            jax.ShapeDtypeStruct((E,), jnp.float32),
            jax.ShapeDtypeStruct((NC * N,), jnp.float32),
        ],
        scratch_types=[
            pltpu.VMEM((N,), jnp.float32),
            pltpu.VMEM((N,), jnp.float32),
            pltpu.VMEM((EPW,), jnp.int32),
            pltpu.VMEM((EPW,), jnp.int32),
            pltpu.VMEM((EPW,), jnp.float32),
            pltpu.VMEM((K,), jnp.int32),
            pltpu.VMEM((32,), jnp.float32),
            pltpu.VMEM_SHARED((N,), jnp.float32),
        ],
    )
    def k(src_hbm, dst_hbm, as_hbm, ad_hbm, m2_hbm, expe_hbm, denomp_hbm,
          as_v, ad_v, srcs, dsts, expv, dst_idx, m2_v, den_sh):
        cid = lax.axis_index("c")
        sid = lax.axis_index("s")
        wid = sid * NC + cid
        base = wid * EPW

        def zfill(i, _):
            expv[pl.ds(i * 16, 16)] = jnp.zeros((16,), jnp.float32)
            return 0
        lax.fori_loop(0, 125, zfill, 0)

        @pl.when(sid < 5)
        def _():
            pltpu.sync_copy(expv.at[pl.ds(0, 2000)],
                            den_sh.at[pl.ds(sid * 2000, 2000)])

        pltpu.sync_copy(as_hbm, as_v)
        pltpu.sync_copy(ad_hbm, ad_v)
        pltpu.sync_copy(src_hbm.at[pl.ds(base, EPW)], srcs)
        pltpu.sync_copy(dst_hbm.at[pl.ds(base, EPW)], dsts)
        pltpu.sync_copy(m2_hbm, m2_v)

        plsc.subcore_barrier()

        mm = m2_v[pl.ds(0, 16)] + m2_v[pl.ds(16, 16)]
        m = jnp.where(mm > 0.0, mm, 0.2 * mm)

        def chunk_body(j, _):
            off = j * K
            for g in range(K // 16):
                sl = pl.ds(off + g * 16, 16)
                d16 = dsts[sl]
                va = plsc.load_gather(as_v, [srcs[sl]])
                vb = plsc.load_gather(ad_v, [d16])
                e = va + vb
                e = jnp.where(e > 0.0, e, 0.2 * e)
                expv[sl] = jnp.exp(e - m)
                dst_idx[pl.ds(g * 16, 16)] = d16
            pltpu.sync_copy(expv.at[pl.ds(off, K)], den_sh.at[dst_idx], add=True)
            return 0
        lax.fori_loop(0, NCHUNK, chunk_body, 0)

        pltpu.sync_copy(expv, expe_hbm.at[pl.ds(base, EPW)])

        plsc.subcore_barrier()

        @pl.when(sid < 10)
        def _():
            stg = expv.at[pl.ds(0, 1000)]
            pltpu.sync_copy(den_sh.at[pl.ds(sid * 1000, 1000)], stg)
            pltpu.sync_copy(stg, denomp_hbm.at[pl.ds(cid * N + sid * 1000, 1000)])

    return k(src, dst, a_src, a_dst, m2.reshape(-1))


def _sc2(src, dst, expe, denomp, h):
    mesh = plsc.VectorSubcoreMesh(core_axis_name="c", subcore_axis_name="s")

    @functools.partial(
        pl.kernel,
        mesh=mesh,
        compiler_params=pltpu.CompilerParams(needs_layout_passes=False),
        out_type=[
            jax.ShapeDtypeStruct((E,), jnp.float32),
            jax.ShapeDtypeStruct((NC * N, D), jnp.float32),
        ],
        scratch_types=[
            pltpu.VMEM((N,), jnp.float32),
            pltpu.VMEM((N,), jnp.float32),
            pltpu.VMEM((K,), jnp.int32),
            pltpu.VMEM((K,), jnp.int32),
            pltpu.VMEM((K,), jnp.float32),
            pltpu.VMEM((K,), jnp.float32),
            pltpu.VMEM((K, D), jnp.float32),
            pltpu.VMEM_SHARED((N, D), jnp.float32),
            pltpu.SemaphoreType.DMA,
        ],
    )
    def k(src_hbm, dst_hbm, expe_hbm, denomp_hbm, h_hbm,
          alpha_hbm, outp_hbm,
          d0_v, d1_v, src_idx, dst_idx, exp_b, al_b, rows, out_sh, sem):
        cid = lax.axis_index("c")
        sid = lax.axis_index("s")
        wid = sid * NC + cid
        base = wid * EPW

        def zfill(i, _):
            for t in range(D // 16):
                rows[i, pl.ds(t * 16, 16)] = jnp.zeros((16,), jnp.float32)
            return 0
        lax.fori_loop(0, 40, zfill, 0)

        @pl.when(sid < 10)
        def _():
            for r in range(25):
                pltpu.sync_copy(rows.at[pl.ds(0, 40)],
                                out_sh.at[pl.ds(sid * 1000 + r * 40, 40)])

        pltpu.sync_copy(denomp_hbm.at[pl.ds(0, N)], d0_v)
        pltpu.sync_copy(denomp_hbm.at[pl.ds(N, N)], d1_v)

        plsc.subcore_barrier()

        def chunk_body(j, _):
            off = base + j * K
            pltpu.sync_copy(src_hbm.at[pl.ds(off, K)], src_idx)
            pltpu.sync_copy(dst_hbm.at[pl.ds(off, K)], dst_idx)
            gat = pltpu.async_copy(h_hbm.at[src_idx], rows, sem)
            pltpu.sync_copy(expe_hbm.at[pl.ds(off, K)], exp_b)
            for g in range(K // 16):
                sl = pl.ds(g * 16, 16)
                d16 = dst_idx[sl]
                den = plsc.load_gather(d0_v, [d16]) + plsc.load_gather(d1_v, [d16])
                al_b[sl] = exp_b[sl] / (den + 1e-16)
            pltpu.sync_copy(al_b, alpha_hbm.at[pl.ds(off, K)])
            gat.wait()
            for g in range(K // 16):
                a16 = al_b[pl.ds(g * 16, 16)]
                for e in range(16):
                    a = a16[e]
                    for t in range(D // 16):
                        sl = pl.ds(t * 16, 16)
                        rows[g * 16 + e, sl] = rows[g * 16 + e, sl] * a
            pltpu.sync_copy(rows, out_sh.at[dst_idx], add=True)
            return 0
        lax.fori_loop(0, NCHUNK, chunk_body, 0)

        plsc.subcore_barrier()

        @pl.when(sid < 10)
        def _():
            for r in range(25):
                stg = rows.at[pl.ds(0, 40)]
                pltpu.sync_copy(out_sh.at[pl.ds(sid * 1000 + r * 40, 40)], stg)
                pltpu.sync_copy(
                    stg, outp_hbm.at[pl.ds(cid * N + sid * 1000 + r * 40, 40)])

    return k(src, dst, expe, denomp, h)


def _tc2_body(p0_ref, p1_ref, b_ref, out_ref):
    out_ref[...] = p0_ref[...] + p1_ref[...] + b_ref[...]


def _tc2(p0, p1, bias):
    blk = 1000
    return pl.pallas_call(
        _tc2_body,
        grid=(N // blk,),
        in_specs=[
            pl.BlockSpec((blk, D), lambda i: (i, 0)),
            pl.BlockSpec((blk, D), lambda i: (i, 0)),
            pl.BlockSpec((1, D), lambda i: (0, 0)),
        ],
        out_specs=pl.BlockSpec((blk, D), lambda i: (i, 0)),
        out_shape=jax.ShapeDtypeStruct((N, D), jnp.float32),
    )(p0, p1, bias.reshape(1, D))


def kernel(x, edge_index, W, att_src, att_dst, bias):
    src = edge_index[0]
    dst = edge_index[1]
    h, a8, m2 = _tc1(x, W, att_src, att_dst)
    expe, denomp = _sc1(src, dst, a8[:, 0], a8[:, 1], m2)
    alpha, outp = _sc2(src, dst, expe, denomp, h)
    out = _tc2(outp[:N], outp[N:], bias)
    return out, edge_index, alpha


# speedup vs baseline: 26.7869x; 1.1279x over previous
import functools

import jax
import jax.numpy as jnp
from jax import lax
from jax.experimental import pallas as pl
from jax.experimental.pallas import tpu as pltpu
from jax.experimental.pallas import tpu_sc as plsc

N = 10000
E = 320000
D = 128

NC = 2
NS = 16
NW = NC * NS
EPW = E // NW
K = 80
NCHUNK = EPW // K


def _tc1_body(x_ref, w_ref, as_ref, ad_ref, h_ref, a8_ref, m2_ref):
    i = pl.program_id(0)
    h = jnp.dot(x_ref[...], w_ref[...], preferred_element_type=jnp.float32)
    h_ref[...] = h
    a_s = jnp.sum(h * as_ref[...], axis=1)
    a_d = jnp.sum(h * ad_ref[...], axis=1)
    z = jnp.zeros_like(a_s)
    a8_ref[...] = jnp.stack([a_s, a_d, z, z, z, z, z, z], axis=1)
    mblk = jnp.stack([jnp.full((16,), jnp.max(a_s)),
                      jnp.full((16,), jnp.max(a_d))], axis=0)

    @pl.when(i == 0)
    def _():
        m2_ref[...] = mblk

    @pl.when(i > 0)
    def _():
        m2_ref[...] = jnp.maximum(m2_ref[...], mblk)


def _tc1(x, W, att_src, att_dst):
    blk = 1000
    return pl.pallas_call(
        _tc1_body,
        grid=(N // blk,),
        in_specs=[
            pl.BlockSpec((blk, D), lambda i: (i, 0)),
            pl.BlockSpec((D, D), lambda i: (0, 0)),
            pl.BlockSpec((1, D), lambda i: (0, 0)),
            pl.BlockSpec((1, D), lambda i: (0, 0)),
        ],
        out_specs=[
            pl.BlockSpec((blk, D), lambda i: (i, 0)),
            pl.BlockSpec((blk, 8), lambda i: (i, 0)),
            pl.BlockSpec((2, 16), lambda i: (0, 0)),
        ],
        out_shape=[
            jax.ShapeDtypeStruct((N, D), jnp.float32),
            jax.ShapeDtypeStruct((N, 8), jnp.float32),
            jax.ShapeDtypeStruct((2, 16), jnp.float32),
        ],
    )(x, W, att_src.reshape(1, D), att_dst.reshape(1, D))


def _sc1(src, dst, a_src, a_dst, m2):
    mesh = plsc.VectorSubcoreMesh(core_axis_name="c", subcore_axis_name="s")

    @functools.partial(
        pl.kernel,
        mesh=mesh,
        compiler_params=pltpu.CompilerParams(needs_layout_passes=False),
        out_type=[
            jax.ShapeDtypeStruct((E,), jnp.float32),
            jax.ShapeDtypeStruct((NC * N,), jnp.float32),
        ],
        scratch_types=[
            pltpu.VMEM((N,), jnp.float32),
            pltpu.VMEM((N,), jnp.float32),
            pltpu.VMEM((EPW,), jnp.int32),
            pltpu.VMEM((EPW,), jnp.int32),
            pltpu.VMEM((EPW,), jnp.float32),
            pltpu.VMEM((K,), jnp.int32),
            pltpu.VMEM((32,), jnp.float32),
            pltpu.VMEM_SHARED((N,), jnp.float32),
        ],
    )
    def k(src_hbm, dst_hbm, as_hbm, ad_hbm, m2_hbm, expe_hbm, denomp_hbm,
          as_v, ad_v, srcs, dsts, expv, dst_idx, m2_v, den_sh):
        cid = lax.axis_index("c")
        sid = lax.axis_index("s")
        wid = sid * NC + cid
        base = wid * EPW

        def zfill(i, _):
            expv[pl.ds(i * 16, 16)] = jnp.zeros((16,), jnp.float32)
            return 0
        lax.fori_loop(0, 125, zfill, 0)

        @pl.when(sid < 5)
        def _():
            pltpu.sync_copy(expv.at[pl.ds(0, 2000)],
                            den_sh.at[pl.ds(sid * 2000, 2000)])

        pltpu.sync_copy(as_hbm, as_v)
        pltpu.sync_copy(ad_hbm, ad_v)
        pltpu.sync_copy(src_hbm.at[pl.ds(base, EPW)], srcs)
        pltpu.sync_copy(dst_hbm.at[pl.ds(base, EPW)], dsts)
        pltpu.sync_copy(m2_hbm, m2_v)

        plsc.subcore_barrier()

        mm = m2_v[pl.ds(0, 16)] + m2_v[pl.ds(16, 16)]
        m = jnp.where(mm > 0.0, mm, 0.2 * mm)

        def chunk_body(j, _):
            off = j * K
            for g in range(K // 16):
                sl = pl.ds(off + g * 16, 16)
                d16 = dsts[sl]
                va = plsc.load_gather(as_v, [srcs[sl]])
                vb = plsc.load_gather(ad_v, [d16])
                e = va + vb
                e = jnp.where(e > 0.0, e, 0.2 * e)
                expv[sl] = jnp.exp(e - m)
                dst_idx[pl.ds(g * 16, 16)] = d16
            pltpu.sync_copy(expv.at[pl.ds(off, K)], den_sh.at[dst_idx], add=True)
            return 0
        lax.fori_loop(0, NCHUNK, chunk_body, 0)

        pltpu.sync_copy(expv, expe_hbm.at[pl.ds(base, EPW)])

        plsc.subcore_barrier()

        @pl.when(sid < 10)
        def _():
            stg = expv.at[pl.ds(0, 1000)]
            pltpu.sync_copy(den_sh.at[pl.ds(sid * 1000, 1000)], stg)
            pltpu.sync_copy(stg, denomp_hbm.at[pl.ds(cid * N + sid * 1000, 1000)])

    return k(src, dst, a_src, a_dst, m2.reshape(-1))


def _sc2(src, dst, expe, denomp, h):
    mesh = plsc.VectorSubcoreMesh(core_axis_name="c", subcore_axis_name="s")

    @functools.partial(
        pl.kernel,
        mesh=mesh,
        compiler_params=pltpu.CompilerParams(needs_layout_passes=False),
        out_type=[
            jax.ShapeDtypeStruct((E,), jnp.float32),
            jax.ShapeDtypeStruct((NC * N, D), jnp.float32),
        ],
        scratch_types=[
            pltpu.VMEM((N,), jnp.float32),
            pltpu.VMEM((EPW,), jnp.float32),
            pltpu.VMEM((K,), jnp.float32),
            pltpu.VMEM((K,), jnp.int32),
            pltpu.VMEM((K,), jnp.int32),
            pltpu.VMEM((K,), jnp.int32),
            pltpu.VMEM((K,), jnp.int32),
            pltpu.VMEM((K, D), jnp.float32),
            pltpu.VMEM((K, D), jnp.float32),
            pltpu.VMEM_SHARED((N, D), jnp.float32),
            pltpu.SemaphoreType.DMA,
            pltpu.SemaphoreType.DMA,
            pltpu.SemaphoreType.DMA,
            pltpu.SemaphoreType.DMA,
            pltpu.SemaphoreType.DMA,
            pltpu.SemaphoreType.DMA,
            pltpu.SemaphoreType.DMA,
            pltpu.SemaphoreType.DMA,
        ],
    )
    def k(src_hbm, dst_hbm, expe_hbm, denomp_hbm, h_hbm,
          alpha_hbm, outp_hbm,
          den_v, expv, al_b,
          siA, siB, diA, diB, rowsA, rowsB, out_sh,
          sgA, sgB, ssA, ssB, slA, slB, sdA, sdB):
        cid = lax.axis_index("c")
        sid = lax.axis_index("s")
        wid = sid * NC + cid
        base = wid * EPW

        def zfill(i, _):
            for t in range(D // 16):
                rowsA[i, pl.ds(t * 16, 16)] = jnp.zeros((16,), jnp.float32)
            return 0
        lax.fori_loop(0, 40, zfill, 0)

        @pl.when(sid < 10)
        def _():
            for r in range(25):
                pltpu.sync_copy(rowsA.at[pl.ds(0, 40)],
                                out_sh.at[pl.ds(sid * 1000 + r * 40, 40)])

        pltpu.sync_copy(denomp_hbm.at[pl.ds(0, N)], den_v)
        pltpu.sync_copy(denomp_hbm.at[pl.ds(N, N)], expv.at[pl.ds(0, N)])

        def dadd(i, _):
            sl = pl.ds(i * 16, 16)
            den_v[sl] = den_v[sl] + expv[sl]
            return 0
        lax.fori_loop(0, N // 16, dadd, 0)
        pltpu.sync_copy(expe_hbm.at[pl.ds(base, EPW)], expv)

        plsc.subcore_barrier()

        dummy = h_hbm.at[pl.ds(0, K)]
        dummy_i = src_hbm.at[pl.ds(0, K)]

        def load_idx(j, si, di, sl_s, sd_s):
            off = base + j * K
            pltpu.async_copy(src_hbm.at[pl.ds(off, K)], si, sl_s)
            pltpu.async_copy(dst_hbm.at[pl.ds(off, K)], di, sd_s)

        def step(j, si_x, di_x, rows_x, sg_x, ss_x, sd_x,
                 si_y, di_y, rows_y, sg_y, ss_y, sl_y, sd_y, gather_next):
            pltpu.make_async_copy(dummy_i, di_x, sd_x).wait()
            off = j * K
            for g in range(K // 16):
                sl16 = pl.ds(g * 16, 16)
                d16 = di_x[sl16]
                den = plsc.load_gather(den_v, [d16])
                a = expv[pl.ds(off + g * 16, 16)] / (den + 1e-16)
                al_b[sl16] = a
                expv[pl.ds(off + g * 16, 16)] = a
            pltpu.make_async_copy(dummy, rows_x, sg_x).wait()
            for g in range(K // 16):
                a16 = al_b[pl.ds(g * 16, 16)]
                for e in range(16):
                    a = a16[e]
                    for t in range(D // 16):
                        sl = pl.ds(t * 16, 16)
                        rows_x[g * 16 + e, sl] = rows_x[g * 16 + e, sl] * a
            @pl.when(j > 0)
            def _():
                pltpu.make_async_copy(dummy, rows_y, ss_y).wait()
            if gather_next:
                load_idx(j + 1, si_y, di_y, sl_y, sd_y)
            pltpu.async_copy(rows_x, out_sh.at[di_x], ss_x, add=True)
            if gather_next:
                pltpu.make_async_copy(dummy_i, si_y, sl_y).wait()
                pltpu.async_copy(h_hbm.at[si_y], rows_y, sg_y)

        load_idx(0, siA, diA, slA, sdA)
        pltpu.make_async_copy(dummy_i, siA, slA).wait()
        pltpu.async_copy(h_hbm.at[siA], rowsA, sgA)

        def pair_body(t, _):
            j0 = t * 2
            step(j0, siA, diA, rowsA, sgA, ssA, sdA,
                 siB, diB, rowsB, sgB, ssB, slB, sdB, True)
            step(j0 + 1, siB, diB, rowsB, sgB, ssB, sdB,
                 siA, diA, rowsA, sgA, ssA, slA, sdA, True)
            return 0
        lax.fori_loop(0, (NCHUNK - 1) // 2, pair_body, 0)

        step(NCHUNK - 1, siA, diA, rowsA, sgA, ssA, sdA,
             siB, diB, rowsB, sgB, ssB, slB, sdB, False)
        pltpu.make_async_copy(dummy, rowsA, ssA).wait()

        pltpu.sync_copy(expv, alpha_hbm.at[pl.ds(base, EPW)])

        plsc.subcore_barrier()

        @pl.when(sid < 10)
        def _():
            for r in range(25):
                stg = rowsA.at[pl.ds(0, 40)]
                pltpu.sync_copy(out_sh.at[pl.ds(sid * 1000 + r * 40, 40)], stg)
                pltpu.sync_copy(
                    stg, outp_hbm.at[pl.ds(cid * N + sid * 1000 + r * 40, 40)])

    return k(src, dst, expe, denomp, h)


def _tc2_body(p0_ref, p1_ref, b_ref, out_ref):
    out_ref[...] = p0_ref[...] + p1_ref[...] + b_ref[...]


def _tc2(p0, p1, bias):
    blk = 1000
    return pl.pallas_call(
        _tc2_body,
        grid=(N // blk,),
        in_specs=[
            pl.BlockSpec((blk, D), lambda i: (i, 0)),
            pl.BlockSpec((blk, D), lambda i: (i, 0)),
            pl.BlockSpec((1, D), lambda i: (0, 0)),
        ],
        out_specs=pl.BlockSpec((blk, D), lambda i: (i, 0)),
        out_shape=jax.ShapeDtypeStruct((N, D), jnp.float32),
    )(p0, p1, bias.reshape(1, D))


def kernel(x, edge_index, W, att_src, att_dst, bias):
    src = edge_index[0]
    dst = edge_index[1]
    h, a8, m2 = _tc1(x, W, att_src, att_dst)
    expe, denomp = _sc1(src, dst, a8[:, 0], a8[:, 1], m2)
    alpha, outp = _sc2(src, dst, expe, denomp, h)
    out = _tc2(outp[:N], outp[N:], bias)
    return out, edge_index, alpha


# speedup vs baseline: 31.3051x; 1.1687x over previous
import functools

import jax
import jax.numpy as jnp
from jax import lax
from jax.experimental import pallas as pl
from jax.experimental.pallas import tpu as pltpu
from jax.experimental.pallas import tpu_sc as plsc

N = 10000
E = 320000
D = 128

NC = 2
NS = 16
NW = NC * NS
EPW = E // NW
K = 80
NCHUNK = EPW // K


def _tc1_body(x_ref, w_ref, as_ref, ad_ref, h_ref, a8_ref, m2_ref):
    i = pl.program_id(0)
    h = jnp.dot(x_ref[...], w_ref[...], preferred_element_type=jnp.float32)
    h_ref[...] = h
    a_s = jnp.sum(h * as_ref[...], axis=1)
    a_d = jnp.sum(h * ad_ref[...], axis=1)
    z = jnp.zeros_like(a_s)
    a8_ref[...] = jnp.stack([a_s, a_d, z, z, z, z, z, z], axis=1)
    mblk = jnp.stack([jnp.full((16,), jnp.max(a_s)),
                      jnp.full((16,), jnp.max(a_d))], axis=0)

    @pl.when(i == 0)
    def _():
        m2_ref[...] = mblk

    @pl.when(i > 0)
    def _():
        m2_ref[...] = jnp.maximum(m2_ref[...], mblk)


def _tc1(x, W, att_src, att_dst):
    blk = 1000
    return pl.pallas_call(
        _tc1_body,
        grid=(N // blk,),
        in_specs=[
            pl.BlockSpec((blk, D), lambda i: (i, 0)),
            pl.BlockSpec((D, D), lambda i: (0, 0)),
            pl.BlockSpec((1, D), lambda i: (0, 0)),
            pl.BlockSpec((1, D), lambda i: (0, 0)),
        ],
        out_specs=[
            pl.BlockSpec((blk, D), lambda i: (i, 0)),
            pl.BlockSpec((blk, 8), lambda i: (i, 0)),
            pl.BlockSpec((2, 16), lambda i: (0, 0)),
        ],
        out_shape=[
            jax.ShapeDtypeStruct((N, D), jnp.float32),
            jax.ShapeDtypeStruct((N, 8), jnp.float32),
            jax.ShapeDtypeStruct((2, 16), jnp.float32),
        ],
    )(x, W, att_src.reshape(1, D), att_dst.reshape(1, D))


def _sc1(src, dst, a_src, a_dst, m2):
    mesh = plsc.VectorSubcoreMesh(core_axis_name="c", subcore_axis_name="s")

    @functools.partial(
        pl.kernel,
        mesh=mesh,
        compiler_params=pltpu.CompilerParams(needs_layout_passes=False),
        out_type=[
            jax.ShapeDtypeStruct((E,), jnp.float32),
            jax.ShapeDtypeStruct((NC * N,), jnp.float32),
        ],
        scratch_types=[
            pltpu.VMEM((N,), jnp.float32),
            pltpu.VMEM((N,), jnp.float32),
            pltpu.VMEM((EPW,), jnp.int32),
            pltpu.VMEM((EPW,), jnp.int32),
            pltpu.VMEM((EPW,), jnp.float32),
            pltpu.VMEM((K,), jnp.int32),
            pltpu.VMEM((32,), jnp.float32),
            pltpu.VMEM_SHARED((N,), jnp.float32),
        ],
    )
    def k(src_hbm, dst_hbm, as_hbm, ad_hbm, m2_hbm, expe_hbm, denomp_hbm,
          as_v, ad_v, srcs, dsts, expv, dst_idx, m2_v, den_sh):
        cid = lax.axis_index("c")
        sid = lax.axis_index("s")
        wid = sid * NC + cid
        base = wid * EPW

        def zfill(i, _):
            expv[pl.ds(i * 16, 16)] = jnp.zeros((16,), jnp.float32)
            return 0
        lax.fori_loop(0, 125, zfill, 0)

        @pl.when(sid < 5)
        def _():
            pltpu.sync_copy(expv.at[pl.ds(0, 2000)],
                            den_sh.at[pl.ds(sid * 2000, 2000)])

        pltpu.sync_copy(as_hbm, as_v)
        pltpu.sync_copy(ad_hbm, ad_v)
        pltpu.sync_copy(src_hbm.at[pl.ds(base, EPW)], srcs)
        pltpu.sync_copy(dst_hbm.at[pl.ds(base, EPW)], dsts)
        pltpu.sync_copy(m2_hbm, m2_v)

        plsc.subcore_barrier()

        mm = m2_v[pl.ds(0, 16)] + m2_v[pl.ds(16, 16)]
        m = jnp.where(mm > 0.0, mm, 0.2 * mm)

        def chunk_body(j, _):
            off = j * K
            for g in range(K // 16):
                sl = pl.ds(off + g * 16, 16)
                d16 = dsts[sl]
                va = plsc.load_gather(as_v, [srcs[sl]])
                vb = plsc.load_gather(ad_v, [d16])
                e = va + vb
                e = jnp.where(e > 0.0, e, 0.2 * e)
                expv[sl] = jnp.exp(e - m)
                dst_idx[pl.ds(g * 16, 16)] = d16
            pltpu.sync_copy(expv.at[pl.ds(off, K)], den_sh.at[dst_idx], add=True)
            return 0
        lax.fori_loop(0, NCHUNK, chunk_body, 0)

        pltpu.sync_copy(expv, expe_hbm.at[pl.ds(base, EPW)])

        plsc.subcore_barrier()

        @pl.when(sid < 10)
        def _():
            stg = expv.at[pl.ds(0, 1000)]
            pltpu.sync_copy(den_sh.at[pl.ds(sid * 1000, 1000)], stg)
            pltpu.sync_copy(stg, denomp_hbm.at[pl.ds(cid * N + sid * 1000, 1000)])

    return k(src, dst, a_src, a_dst, m2.reshape(-1))


def _sc2(src, dst, expe, denomp, h):
    mesh = plsc.VectorSubcoreMesh(core_axis_name="c", subcore_axis_name="s")

    @functools.partial(
        pl.kernel,
        mesh=mesh,
        compiler_params=pltpu.CompilerParams(needs_layout_passes=False),
        out_type=[
            jax.ShapeDtypeStruct((E,), jnp.float32),
            jax.ShapeDtypeStruct((NC * N, D), jnp.float32),
        ],
        scratch_types=[
            pltpu.VMEM((N,), jnp.float32),
            pltpu.VMEM((EPW,), jnp.float32),
            pltpu.VMEM((K,), jnp.float32),
            pltpu.VMEM((K,), jnp.int32),
            pltpu.VMEM((K,), jnp.int32),
            pltpu.VMEM((K,), jnp.int32),
            pltpu.VMEM((K,), jnp.int32),
            pltpu.VMEM((K, D), jnp.float32),
            pltpu.VMEM((K, D), jnp.float32),
            pltpu.VMEM_SHARED((N, D), jnp.float32),
            pltpu.SemaphoreType.DMA,
            pltpu.SemaphoreType.DMA,
            pltpu.SemaphoreType.DMA,
            pltpu.SemaphoreType.DMA,
            pltpu.SemaphoreType.DMA,
            pltpu.SemaphoreType.DMA,
            pltpu.SemaphoreType.DMA,
            pltpu.SemaphoreType.DMA,
        ],
    )
    def k(src_hbm, dst_hbm, expe_hbm, denomp_hbm, h_hbm,
          alpha_hbm, outp_hbm,
          den_v, expv, al_b,
          siA, siB, diA, diB, rowsA, rowsB, out_sh,
          sgA, sgB, ssA, ssB, slA, slB, sdA, sdB):
        cid = lax.axis_index("c")
        sid = lax.axis_index("s")
        wid = sid * NC + cid
        base = wid * EPW

        def zfill(i, _):
            for t in range(D // 16):
                rowsA[i, pl.ds(t * 16, 16)] = jnp.zeros((16,), jnp.float32)
            return 0
        lax.fori_loop(0, 40, zfill, 0)

        @pl.when(sid < 10)
        def _():
            for r in range(25):
                pltpu.sync_copy(rowsA.at[pl.ds(0, 40)],
                                out_sh.at[pl.ds(sid * 1000 + r * 40, 40)])

        pltpu.sync_copy(denomp_hbm.at[pl.ds(0, N)], den_v)
        pltpu.sync_copy(denomp_hbm.at[pl.ds(N, N)], expv.at[pl.ds(0, N)])

        def dadd(i, _):
            sl = pl.ds(i * 16, 16)
            den_v[sl] = den_v[sl] + expv[sl]
            return 0
        lax.fori_loop(0, N // 16, dadd, 0)
        pltpu.sync_copy(expe_hbm.at[pl.ds(base, EPW)], expv)

        plsc.subcore_barrier()

        dummy = h_hbm.at[pl.ds(0, K)]
        dummy_i = src_hbm.at[pl.ds(0, K)]

        def load_idx(j, si, di, sl_s, sd_s):
            off = base + j * K
            pltpu.async_copy(src_hbm.at[pl.ds(off, K)], si, sl_s)
            pltpu.async_copy(dst_hbm.at[pl.ds(off, K)], di, sd_s)

        def step(j, si_x, di_x, rows_x, sg_x, ss_x, sd_x,
                 si_y, di_y, rows_y, sg_y, ss_y, sl_y, sd_y, gather_next):
            pltpu.make_async_copy(dummy_i, di_x, sd_x).wait()
            off = j * K
            for g in range(K // 16):
                sl16 = pl.ds(g * 16, 16)
                d16 = di_x[sl16]
                den = plsc.load_gather(den_v, [d16])
                a = expv[pl.ds(off + g * 16, 16)] / (den + 1e-16)
                al_b[sl16] = a
                expv[pl.ds(off + g * 16, 16)] = a
            pltpu.make_async_copy(dummy, rows_x, sg_x).wait()
            @pl.when(j > 0)
            def _():
                pltpu.make_async_copy(dummy, rows_y, ss_y).wait()
            if gather_next:
                load_idx(j + 1, si_y, di_y, sl_y, sd_y)
                pltpu.make_async_copy(dummy_i, si_y, sl_y).wait()
                pltpu.async_copy(h_hbm.at[si_y], rows_y, sg_y)
            for g in range(K // 16):
                a16 = al_b[pl.ds(g * 16, 16)]
                for e in range(16):
                    a = a16[e]
                    for t in range(D // 16):
                        sl = pl.ds(t * 16, 16)
                        rows_x[g * 16 + e, sl] = rows_x[g * 16 + e, sl] * a
            pltpu.async_copy(rows_x, out_sh.at[di_x], ss_x, add=True)

        load_idx(0, siA, diA, slA, sdA)
        pltpu.make_async_copy(dummy_i, siA, slA).wait()
        pltpu.async_copy(h_hbm.at[siA], rowsA, sgA)

        def pair_body(t, _):
            j0 = t * 2
            step(j0, siA, diA, rowsA, sgA, ssA, sdA,
                 siB, diB, rowsB, sgB, ssB, slB, sdB, True)
            step(j0 + 1, siB, diB, rowsB, sgB, ssB, sdB,
                 siA, diA, rowsA, sgA, ssA, slA, sdA, True)
            return 0
        lax.fori_loop(0, (NCHUNK - 1) // 2, pair_body, 0)

        step(NCHUNK - 1, siA, diA, rowsA, sgA, ssA, sdA,
             siB, diB, rowsB, sgB, ssB, slB, sdB, False)
        pltpu.make_async_copy(dummy, rowsA, ssA).wait()

        pltpu.sync_copy(expv, alpha_hbm.at[pl.ds(base, EPW)])

        plsc.subcore_barrier()

        @pl.when(sid < 10)
        def _():
            for r in range(25):
                stg = rowsA.at[pl.ds(0, 40)]
                pltpu.sync_copy(out_sh.at[pl.ds(sid * 1000 + r * 40, 40)], stg)
                pltpu.sync_copy(
                    stg, outp_hbm.at[pl.ds(cid * N + sid * 1000 + r * 40, 40)])

    return k(src, dst, expe, denomp, h)


def _tc2_body(p0_ref, p1_ref, b_ref, out_ref):
    out_ref[...] = p0_ref[...] + p1_ref[...] + b_ref[...]


def _tc2(p0, p1, bias):
    blk = 1000
    return pl.pallas_call(
        _tc2_body,
        grid=(N // blk,),
        in_specs=[
            pl.BlockSpec((blk, D), lambda i: (i, 0)),
            pl.BlockSpec((blk, D), lambda i: (i, 0)),
            pl.BlockSpec((1, D), lambda i: (0, 0)),
        ],
        out_specs=pl.BlockSpec((blk, D), lambda i: (i, 0)),
        out_shape=jax.ShapeDtypeStruct((N, D), jnp.float32),
    )(p0, p1, bias.reshape(1, D))


def kernel(x, edge_index, W, att_src, att_dst, bias):
    src = edge_index[0]
    dst = edge_index[1]
    h, a8, m2 = _tc1(x, W, att_src, att_dst)
    expe, denomp = _sc1(src, dst, a8[:, 0], a8[:, 1], m2)
    alpha, outp = _sc2(src, dst, expe, denomp, h)
    out = _tc2(outp[:N], outp[N:], bias)
    return out, edge_index, alpha


# speedup vs baseline: 33.0842x; 1.0568x over previous
import functools

import jax
import jax.numpy as jnp
from jax import lax
from jax.experimental import pallas as pl
from jax.experimental.pallas import tpu as pltpu
from jax.experimental.pallas import tpu_sc as plsc

N = 10000
E = 320000
D = 128

NC = 2
NS = 16
NW = NC * NS
EPW = E // NW
K = 80
NCHUNK = EPW // K


def _tc1_body(x_ref, w_ref, as_ref, ad_ref, h_ref, a8_ref, m2_ref):
    i = pl.program_id(0)
    h = jnp.dot(x_ref[...], w_ref[...], preferred_element_type=jnp.float32)
    h_ref[...] = h
    a_s = jnp.sum(h * as_ref[...], axis=1)
    a_d = jnp.sum(h * ad_ref[...], axis=1)
    z = jnp.zeros_like(a_s)
    a8_ref[...] = jnp.stack([a_s, a_d, z, z, z, z, z, z], axis=1)
    mblk = jnp.stack([jnp.full((16,), jnp.max(a_s)),
                      jnp.full((16,), jnp.max(a_d))], axis=0)

    @pl.when(i == 0)
    def _():
        m2_ref[...] = mblk

    @pl.when(i > 0)
    def _():
        m2_ref[...] = jnp.maximum(m2_ref[...], mblk)


def _tc1(x, W, att_src, att_dst):
    blk = 1000
    return pl.pallas_call(
        _tc1_body,
        grid=(N // blk,),
        in_specs=[
            pl.BlockSpec((blk, D), lambda i: (i, 0)),
            pl.BlockSpec((D, D), lambda i: (0, 0)),
            pl.BlockSpec((1, D), lambda i: (0, 0)),
            pl.BlockSpec((1, D), lambda i: (0, 0)),
        ],
        out_specs=[
            pl.BlockSpec((blk, D), lambda i: (i, 0)),
            pl.BlockSpec((blk, 8), lambda i: (i, 0)),
            pl.BlockSpec((2, 16), lambda i: (0, 0)),
        ],
        out_shape=[
            jax.ShapeDtypeStruct((N, D), jnp.float32),
            jax.ShapeDtypeStruct((N, 8), jnp.float32),
            jax.ShapeDtypeStruct((2, 16), jnp.float32),
        ],
    )(x, W, att_src.reshape(1, D), att_dst.reshape(1, D))


def _sc1(src, dst, a_src, a_dst, m2):
    mesh = plsc.VectorSubcoreMesh(core_axis_name="c", subcore_axis_name="s")

    @functools.partial(
        pl.kernel,
        mesh=mesh,
        compiler_params=pltpu.CompilerParams(needs_layout_passes=False),
        out_type=[
            jax.ShapeDtypeStruct((E,), jnp.float32),
            jax.ShapeDtypeStruct((NC * N,), jnp.float32),
        ],
        scratch_types=[
            pltpu.VMEM((N,), jnp.float32),
            pltpu.VMEM((N,), jnp.float32),
            pltpu.VMEM((EPW,), jnp.int32),
            pltpu.VMEM((EPW,), jnp.int32),
            pltpu.VMEM((EPW,), jnp.float32),
            pltpu.VMEM((K,), jnp.int32),
            pltpu.VMEM((32,), jnp.float32),
            pltpu.VMEM_SHARED((N,), jnp.float32),
        ],
    )
    def k(src_hbm, dst_hbm, as_hbm, ad_hbm, m2_hbm, expe_hbm, denomp_hbm,
          as_v, ad_v, srcs, dsts, expv, dst_idx, m2_v, den_sh):
        cid = lax.axis_index("c")
        sid = lax.axis_index("s")
        wid = sid * NC + cid
        base = wid * EPW

        def zfill(i, _):
            expv[pl.ds(i * 16, 16)] = jnp.zeros((16,), jnp.float32)
            return 0
        lax.fori_loop(0, 125, zfill, 0)

        @pl.when(sid < 5)
        def _():
            pltpu.sync_copy(expv.at[pl.ds(0, 2000)],
                            den_sh.at[pl.ds(sid * 2000, 2000)])

        pltpu.sync_copy(as_hbm, as_v)
        pltpu.sync_copy(ad_hbm, ad_v)
        pltpu.sync_copy(src_hbm.at[pl.ds(base, EPW)], srcs)
        pltpu.sync_copy(dst_hbm.at[pl.ds(base, EPW)], dsts)
        pltpu.sync_copy(m2_hbm, m2_v)

        plsc.subcore_barrier()

        mm = m2_v[pl.ds(0, 16)] + m2_v[pl.ds(16, 16)]
        m = jnp.where(mm > 0.0, mm, 0.2 * mm)

        def chunk_body(j, _):
            off = j * K
            for g in range(K // 16):
                sl = pl.ds(off + g * 16, 16)
                d16 = dsts[sl]
                va = plsc.load_gather(as_v, [srcs[sl]])
                vb = plsc.load_gather(ad_v, [d16])
                e = va + vb
                e = jnp.where(e > 0.0, e, 0.2 * e)
                expv[sl] = jnp.exp(e - m)
                dst_idx[pl.ds(g * 16, 16)] = d16
            pltpu.sync_copy(expv.at[pl.ds(off, K)], den_sh.at[dst_idx], add=True)
            return 0
        lax.fori_loop(0, NCHUNK, chunk_body, 0)

        pltpu.sync_copy(expv, expe_hbm.at[pl.ds(base, EPW)])

        plsc.subcore_barrier()

        @pl.when(sid < 10)
        def _():
            stg = expv.at[pl.ds(0, 1000)]
            pltpu.sync_copy(den_sh.at[pl.ds(sid * 1000, 1000)], stg)
            pltpu.sync_copy(stg, denomp_hbm.at[pl.ds(cid * N + sid * 1000, 1000)])

    return k(src, dst, a_src, a_dst, m2.reshape(-1))


def _tcd_body(p_ref, out_ref):
    p = p_ref[...]
    out_ref[...] = p[0:1, :] + p[1:2, :]


def _tcd(denomp):
    out = pl.pallas_call(
        _tcd_body,
        out_shape=jax.ShapeDtypeStruct((1, N), jnp.float32),
    )(denomp.reshape(2, N))
    return out.reshape(N)


def _sc2(src, dst, expe, denom, h):
    mesh = plsc.VectorSubcoreMesh(core_axis_name="c", subcore_axis_name="s")

    @functools.partial(
        pl.kernel,
        mesh=mesh,
        compiler_params=pltpu.CompilerParams(needs_layout_passes=False),
        out_type=[
            jax.ShapeDtypeStruct((E,), jnp.float32),
            jax.ShapeDtypeStruct((NC * N, D), jnp.float32),
        ],
        scratch_types=[
            pltpu.VMEM((N,), jnp.float32),
            pltpu.VMEM((K,), jnp.int32),
            pltpu.VMEM((K,), jnp.int32),
            pltpu.VMEM((K,), jnp.int32),
            pltpu.VMEM((K,), jnp.int32),
            pltpu.VMEM((K,), jnp.int32),
            pltpu.VMEM((K,), jnp.int32),
            pltpu.VMEM((K,), jnp.float32),
            pltpu.VMEM((K,), jnp.float32),
            pltpu.VMEM((K,), jnp.float32),
            pltpu.VMEM((K,), jnp.float32),
            pltpu.VMEM((K,), jnp.float32),
            pltpu.VMEM((K,), jnp.float32),
            pltpu.VMEM((K, D), jnp.float32),
            pltpu.VMEM((K, D), jnp.float32),
            pltpu.VMEM((K, D), jnp.float32),
            pltpu.VMEM_SHARED((N, D), jnp.float32),
            pltpu.SemaphoreType.DMA,
            pltpu.SemaphoreType.DMA,
            pltpu.SemaphoreType.DMA,
            pltpu.SemaphoreType.DMA,
            pltpu.SemaphoreType.DMA,
            pltpu.SemaphoreType.DMA,
            pltpu.SemaphoreType.DMA,
            pltpu.SemaphoreType.DMA,
            pltpu.SemaphoreType.DMA,
            pltpu.SemaphoreType.DMA,
            pltpu.SemaphoreType.DMA,
            pltpu.SemaphoreType.DMA,
            pltpu.SemaphoreType.DMA,
            pltpu.SemaphoreType.DMA,
            pltpu.SemaphoreType.DMA,
            pltpu.SemaphoreType.DMA,
            pltpu.SemaphoreType.DMA,
            pltpu.SemaphoreType.DMA,
        ],
    )
    def k(src_hbm, dst_hbm, expe_hbm, den_hbm, h_hbm,
          alpha_hbm, outp_hbm,
          den_v,
          si0, si1, si2, di0, di1, di2, ex0, ex1, ex2, al0, al1, al2,
          rw0, rw1, rw2, out_sh,
          sg0, sg1, sg2, ss0, ss1, ss2, sl0, sl1, sl2,
          sd0, sd1, sd2, se0, se1, se2, sa0, sa1, sa2):
        cid = lax.axis_index("c")
        sid = lax.axis_index("s")
        wid = sid * NC + cid
        base = wid * EPW

        SI = (si0, si1, si2)
        DI = (di0, di1, di2)
        EX = (ex0, ex1, ex2)
        AL = (al0, al1, al2)
        RW = (rw0, rw1, rw2)
        SG = (sg0, sg1, sg2)
        SS = (ss0, ss1, ss2)
        SL = (sl0, sl1, sl2)
        SD = (sd0, sd1, sd2)
        SE = (se0, se1, se2)
        SA = (sa0, sa1, sa2)

        def zfill(i, _):
            for t in range(D // 16):
                rw0[i, pl.ds(t * 16, 16)] = jnp.zeros((16,), jnp.float32)
            return 0
        lax.fori_loop(0, 40, zfill, 0)

        @pl.when(sid < 10)
        def _():
            for r in range(25):
                pltpu.sync_copy(rw0.at[pl.ds(0, 40)],
                                out_sh.at[pl.ds(sid * 1000 + r * 40, 40)])

        pltpu.sync_copy(den_hbm, den_v)

        plsc.subcore_barrier()

        dummy_f = h_hbm.at[pl.ds(0, K)]
        dummy_i = src_hbm.at[pl.ds(0, K)]
        dummy_e = expe_hbm.at[pl.ds(0, K)]

        def load_chunk(j, s):
            off = base + j * K
            pltpu.async_copy(src_hbm.at[pl.ds(off, K)], SI[s], SL[s])
            pltpu.async_copy(dst_hbm.at[pl.ds(off, K)], DI[s], SD[s])
            pltpu.async_copy(expe_hbm.at[pl.ds(off, K)], EX[s], SE[s])

        def step(j, x, gather_next):
            z = (x + 1) % 3
            pltpu.make_async_copy(dummy_i, DI[x], SD[x]).wait()
            pltpu.make_async_copy(dummy_e, EX[x], SE[x]).wait()

            @pl.when(j > 2)
            def _():
                pltpu.make_async_copy(dummy_e, AL[x], SA[x]).wait()
            for g in range(K // 16):
                sl16 = pl.ds(g * 16, 16)
                d16 = DI[x][sl16]
                den = plsc.load_gather(den_v, [d16])
                AL[x][sl16] = EX[x][sl16] / (den + 1e-16)
            pltpu.async_copy(AL[x], alpha_hbm.at[pl.ds(base + j * K, K)],
                             SA[x])

            @pl.when(j > 1)
            def _():
                pltpu.make_async_copy(dummy_f, RW[z], SS[z]).wait()
            if gather_next:
                load_chunk(j + 1, z)
            pltpu.make_async_copy(dummy_f, RW[x], SG[x]).wait()
            if gather_next:
                pltpu.make_async_copy(dummy_i, SI[z], SL[z]).wait()
                pltpu.async_copy(h_hbm.at[SI[z]], RW[z], SG[z])
            for g in range(K // 16):
                a16 = AL[x][pl.ds(g * 16, 16)]
                for e in range(16):
                    a = a16[e]
                    for t in range(D // 16):
                        sl = pl.ds(t * 16, 16)
                        RW[x][g * 16 + e, sl] = RW[x][g * 16 + e, sl] * a
            pltpu.async_copy(RW[x], out_sh.at[DI[x]], SS[x], add=True)

        load_chunk(0, 0)
        pltpu.make_async_copy(dummy_i, si0, sl0).wait()
        pltpu.async_copy(h_hbm.at[si0], rw0, sg0)

        def tri_body(t, _):
            j0 = t * 3
            step(j0, 0, True)
            step(j0 + 1, 1, True)
            step(j0 + 2, 2, True)
            return 0
        lax.fori_loop(0, NCHUNK // 3, tri_body, 0)

        step(NCHUNK - 2, 0, True)
        step(NCHUNK - 1, 1, False)

        pltpu.make_async_copy(dummy_f, rw0, ss0).wait()
        pltpu.make_async_copy(dummy_f, rw1, ss1).wait()
        pltpu.make_async_copy(dummy_e, al2, sa2).wait()
        pltpu.make_async_copy(dummy_e, al0, sa0).wait()
        pltpu.make_async_copy(dummy_e, al1, sa1).wait()

        plsc.subcore_barrier()

        @pl.when(sid < 10)
        def _():
            for r in range(25):
                stg = rw0.at[pl.ds(0, 40)]
                pltpu.sync_copy(out_sh.at[pl.ds(sid * 1000 + r * 40, 40)], stg)
                pltpu.sync_copy(
                    stg, outp_hbm.at[pl.ds(cid * N + sid * 1000 + r * 40, 40)])

    return k(src, dst, expe, denom, h)


def _tc2_body(p0_ref, p1_ref, b_ref, out_ref):
    out_ref[...] = p0_ref[...] + p1_ref[...] + b_ref[...]


def _tc2(p0, p1, bias):
    blk = 1000
    return pl.pallas_call(
        _tc2_body,
        grid=(N // blk,),
        in_specs=[
            pl.BlockSpec((blk, D), lambda i: (i, 0)),
            pl.BlockSpec((blk, D), lambda i: (i, 0)),
            pl.BlockSpec((1, D), lambda i: (0, 0)),
        ],
        out_specs=pl.BlockSpec((blk, D), lambda i: (i, 0)),
        out_shape=jax.ShapeDtypeStruct((N, D), jnp.float32),
    )(p0, p1, bias.reshape(1, D))


def kernel(x, edge_index, W, att_src, att_dst, bias):
    src = edge_index[0]
    dst = edge_index[1]
    h, a8, m2 = _tc1(x, W, att_src, att_dst)
    expe, denomp = _sc1(src, dst, a8[:, 0], a8[:, 1], m2)
    denom = _tcd(denomp)
    alpha, outp = _sc2(src, dst, expe, denom, h)
    out = _tc2(outp[:N], outp[N:], bias)
    return out, edge_index, alpha
